# baseline (speedup 1.0000x reference)
import jax
import jax.numpy as jnp
from jax import lax
from jax.experimental import pallas as pl
from jax.experimental.pallas import tpu as pltpu
from jax.experimental.pallas import tpu_sc as plsc

NUM_CORES = 2
NUM_SUBCORES = 16
NUM_WORKERS = NUM_CORES * NUM_SUBCORES
LANES = 16

BATCH = 16384
K = 64
V_CNT = 100000
N_SLABS = 3 * K
SLABS_PER_W = N_SLABS // NUM_WORKERS
HALF_A = 50048
HALF_B = V_CNT - HALF_A
ECHUNK = 2048
N_ECHUNKS = BATCH // ECHUNK
N_PER = BATCH // NUM_WORKERS


def _extract_body(xs_hbm, xr_hbm, xw_hbm, s_hbm, r_hbm, w_hbm, v_hbm,
                  slab_a, slab_bb, idx_p, idx_q, out_b,
                  sem_s, sem_b, sem_p, sem_q, sem_o):
  wid = lax.axis_index("s") * NUM_CORES + lax.axis_index("c")
  zero16 = jnp.zeros((LANES,), jnp.int32)
  x_tabs = (xs_hbm, xr_hbm, xw_hbm)
  tabs = (s_hbm, r_hbm, w_hbm)
  idx_bufs = (idx_p, idx_q)
  idx_sems = (sem_p, sem_q)

  halves = (slab_a, slab_bb)
  half_sems = (sem_s, sem_b)
  lane = lax.iota(jnp.int32, LANES)

  def half_dma(i, h):
    k_rt = (i * NUM_WORKERS + wid) % K
    off = 0 if h == 0 else HALF_A
    n = HALF_A if h == 0 else HALF_B
    return pltpu.async_copy(
        tabs[i // 2].at[pl.ds(k_rt, 1), pl.ds(off, n)], halves[h],
        half_sems[h])

  def extract_pass(i, h):
    x_hbm = x_tabs[i // 2]
    buf = halves[h]
    ci = pltpu.async_copy(x_hbm.at[pl.ds(0, ECHUNK)], idx_bufs[0],
                          idx_sems[0])
    for c in range(N_ECHUNKS):
      ci.wait()
      if c + 1 < N_ECHUNKS:
        nb = (c + 1) % 2
        ci = pltpu.async_copy(
            x_hbm.at[pl.ds((c + 1) * ECHUNK, ECHUNK)], idx_bufs[nb],
            idx_sems[nb])
      ib = idx_bufs[c % 2]

      def chunk_part(u, _, c=c, ib=ib, buf=buf, h=h):
        for g8 in range(8):
          g = u * 8 + g8
          iv = ib[pl.ds(g * LANES, LANES)]
          if h == 0:
            m = iv < HALF_A
            loc = iv
          else:
            m = iv >= HALF_A
            loc = jnp.where(m, iv - HALF_A, 0)
          vals = plsc.load_gather(buf, [zero16, loc], mask=m)
          pos = c * ECHUNK + g * LANES + lane
          plsc.store_scatter(out_b, [zero16, pos], vals, mask=m)
        return 0

      lax.fori_loop(0, ECHUNK // LANES // 8, chunk_part, 0)

  co = None
  ca = half_dma(0, 0)
  for i in range(SLABS_PER_W):
    cb = half_dma(i, 1)
    ca.wait()
    if co is not None:
      co.wait()
    extract_pass(i, 0)
    cb.wait()
    if i + 1 < SLABS_PER_W:
      ca = half_dma(i + 1, 0)
    extract_pass(i, 1)
    row = i * NUM_WORKERS + wid
    co = pltpu.async_copy(out_b, v_hbm.at[pl.ds(row, 1), pl.ds(0, BATCH)],
                          sem_o)
  co.wait()


def _reduce_body(v_hbm, out_hbm, v_buf, out_v, sem):
  wid = lax.axis_index("s") * NUM_CORES + lax.axis_index("c")
  base = wid * N_PER
  pltpu.async_copy(v_hbm.at[pl.ds(0, N_SLABS), pl.ds(base, N_PER)],
                   v_buf, sem).wait()

  def group_body(g, _):
    acc = jnp.zeros((LANES,), jnp.float32)

    def k_body(k8, acc):
      for kk in range(8):
        k = k8 * 8 + kk
        sv = v_buf[0 * K + k, pl.ds(g * LANES, LANES)]
        rv = v_buf[1 * K + k, pl.ds(g * LANES, LANES)]
        wv = v_buf[2 * K + k, pl.ds(g * LANES, LANES)]
        acc = acc + (sv + rv) * wv
      return acc

    acc = lax.fori_loop(0, K // 8, k_body, acc)
    out_v[pl.ds(g * LANES, LANES)] = 1.0 / (1.0 + jnp.exp(-acc))
    return 0

  lax.fori_loop(0, N_PER // LANES, group_body, 0)
  pltpu.sync_copy(out_v, out_hbm.at[pl.ds(base, N_PER)])


@jax.jit
def kernel(X, s_table, r_table, w_table):
  mesh = plsc.VectorSubcoreMesh(core_axis_name="c", subcore_axis_name="s")
  extract = pl.kernel(
      _extract_body,
      out_type=jax.ShapeDtypeStruct((N_SLABS, BATCH), jnp.float32),
      mesh=mesh,
      scratch_types=[
          pltpu.VMEM((1, HALF_A), jnp.float32),
          pltpu.VMEM((1, HALF_B), jnp.float32),
          pltpu.VMEM((ECHUNK,), jnp.int32),
          pltpu.VMEM((ECHUNK,), jnp.int32),
          pltpu.VMEM((1, BATCH), jnp.float32),
          pltpu.SemaphoreType.DMA,
          pltpu.SemaphoreType.DMA,
          pltpu.SemaphoreType.DMA,
          pltpu.SemaphoreType.DMA,
          pltpu.SemaphoreType.DMA,
      ],
      compiler_params=pltpu.CompilerParams(needs_layout_passes=False),
  )
  reduce = pl.kernel(
      _reduce_body,
      out_type=jax.ShapeDtypeStruct((BATCH,), jnp.float32),
      mesh=mesh,
      scratch_types=[
          pltpu.VMEM((N_SLABS, N_PER), jnp.float32),
          pltpu.VMEM((N_PER,), jnp.float32),
          pltpu.SemaphoreType.DMA,
      ],
      compiler_params=pltpu.CompilerParams(needs_layout_passes=False),
  )
  v = extract(X[:, 0], X[:, 1], X[:, 2],
              s_table.T, r_table.T, w_table.T)
  return reduce(v)


# speedup vs baseline: 1.8765x; 1.8765x over previous
import jax
import jax.numpy as jnp
from jax import lax
from jax.experimental import pallas as pl
from jax.experimental.pallas import tpu as pltpu
from jax.experimental.pallas import tpu_sc as plsc

NUM_CORES = 2
NUM_SUBCORES = 16
NUM_WORKERS = NUM_CORES * NUM_SUBCORES
LANES = 16

BATCH = 16384
K = 64
V_CNT = 100000
N_SLABS = 3 * K
SLABS_PER_W = N_SLABS // NUM_WORKERS
HALF_A = 50048
HALF_B = V_CNT - HALF_A
ECHUNK = 4096
N_ECHUNKS = BATCH // ECHUNK
N_PER = BATCH // NUM_WORKERS


def _extract_body(xs_hbm, xr_hbm, xw_hbm, s_hbm, r_hbm, w_hbm, v_hbm,
                  slab_b, idx_p, idx_q, out_b,
                  sem_s, sem_p, sem_q, sem_o):
  wid = lax.axis_index("s") * NUM_CORES + lax.axis_index("c")
  zero16 = jnp.zeros((LANES,), jnp.int32)
  x_tabs = (xs_hbm, xr_hbm, xw_hbm)
  tabs = (s_hbm, r_hbm, w_hbm)
  idx_bufs = (idx_p, idx_q)
  idx_sems = (sem_p, sem_q)

  co = None
  for i in range(SLABS_PER_W):
    k_rt = (i * NUM_WORKERS + wid) % K
    cs = pltpu.async_copy(
        tabs[i // 2].at[pl.ds(k_rt, 1), pl.ds(0, V_CNT)], slab_b, sem_s)
    x_hbm = x_tabs[i // 2]
    ci = pltpu.async_copy(x_hbm.at[pl.ds(0, ECHUNK)], idx_bufs[0],
                          idx_sems[0])
    cs.wait()
    if co is not None:
      co.wait()

    for c in range(N_ECHUNKS):
      ci.wait()
      if c + 1 < N_ECHUNKS:
        nb = (c + 1) % 2
        ci = pltpu.async_copy(
            x_hbm.at[pl.ds((c + 1) * ECHUNK, ECHUNK)], idx_bufs[nb],
            idx_sems[nb])
      ib = idx_bufs[c % 2]

      def chunk_part(u, _, c=c, ib=ib):
        for g8 in range(16):
          g = u * 16 + g8
          iv = ib[pl.ds(g * LANES, LANES)]
          vals = plsc.load_gather(slab_b, [zero16, iv])
          out_b[0, pl.ds(c * ECHUNK + g * LANES, LANES)] = vals
        return 0

      lax.fori_loop(0, ECHUNK // LANES // 16, chunk_part, 0)

    row = i * NUM_WORKERS + wid
    co = pltpu.async_copy(out_b, v_hbm.at[pl.ds(row, 1), pl.ds(0, BATCH)],
                          sem_o)
  co.wait()


def _reduce_body(v_hbm, out_hbm, v_buf, out_v, sem):
  wid = lax.axis_index("s") * NUM_CORES + lax.axis_index("c")
  base = wid * N_PER
  pltpu.async_copy(v_hbm.at[pl.ds(0, N_SLABS), pl.ds(base, N_PER)],
                   v_buf, sem).wait()

  def group_body(g, _):
    acc = jnp.zeros((LANES,), jnp.float32)

    def k_body(k8, acc):
      for kk in range(8):
        k = k8 * 8 + kk
        sv = v_buf[0 * K + k, pl.ds(g * LANES, LANES)]
        rv = v_buf[1 * K + k, pl.ds(g * LANES, LANES)]
        wv = v_buf[2 * K + k, pl.ds(g * LANES, LANES)]
        acc = acc + (sv + rv) * wv
      return acc

    acc = lax.fori_loop(0, K // 8, k_body, acc)
    out_v[pl.ds(g * LANES, LANES)] = 1.0 / (1.0 + jnp.exp(-acc))
    return 0

  lax.fori_loop(0, N_PER // LANES, group_body, 0)
  pltpu.sync_copy(out_v, out_hbm.at[pl.ds(base, N_PER)])


@jax.jit
def kernel(X, s_table, r_table, w_table):
  mesh = plsc.VectorSubcoreMesh(core_axis_name="c", subcore_axis_name="s")
  extract = pl.kernel(
      _extract_body,
      out_type=jax.ShapeDtypeStruct((N_SLABS, BATCH), jnp.float32),
      mesh=mesh,
      scratch_types=[
          pltpu.VMEM((1, V_CNT), jnp.float32),
          pltpu.VMEM((ECHUNK,), jnp.int32),
          pltpu.VMEM((ECHUNK,), jnp.int32),
          pltpu.VMEM((1, BATCH), jnp.float32),
          pltpu.SemaphoreType.DMA,
          pltpu.SemaphoreType.DMA,
          pltpu.SemaphoreType.DMA,
          pltpu.SemaphoreType.DMA,
      ],
      compiler_params=pltpu.CompilerParams(needs_layout_passes=False),
  )
  reduce = pl.kernel(
      _reduce_body,
      out_type=jax.ShapeDtypeStruct((BATCH,), jnp.float32),
      mesh=mesh,
      scratch_types=[
          pltpu.VMEM((N_SLABS, N_PER), jnp.float32),
          pltpu.VMEM((N_PER,), jnp.float32),
          pltpu.SemaphoreType.DMA,
      ],
      compiler_params=pltpu.CompilerParams(needs_layout_passes=False),
  )
  v = extract(X[:, 0], X[:, 1], X[:, 2],
              s_table.T, r_table.T, w_table.T)
  return reduce(v)
